# Initial kernel scaffold; baseline (speedup 1.0000x reference)
import jax
import jax.numpy as jnp
from jax.experimental import pallas as pl


def kernel(u_node, v_node, negative_nodes, emb_u, emb_v, emb_com):
    raise NotImplementedError("write your pallas kernel here")


# speedup vs baseline: 2.1150x; 2.1150x over previous
import functools
import jax
import jax.numpy as jnp
from jax import lax
from jax.experimental import pallas as pl
from jax.experimental.pallas import tpu as pltpu
from jax.experimental.pallas import tpu_sc as plsc

B = 16384
D = 128
NEG = 10
K = 64
GAMMA = 0.01

NC = 2
NS = 16
NW = NC * NS
BW = B // NW

UCH = 128
NCH = 64
NROWS = NCH * NEG
NSEG = NROWS // 128
NEG_ROWS_PER_W = BW * NEG // 128


def _sc_gather_body(uidx_hbm, vidx_hbm, nidx_hbm, emb_u_hbm, emb_v_hbm,
                    u_out, v_out, ns_out,
                    idx_v, rows_v, nidx_v0, nidx_v1, nidx_v2, nidx_v3,
                    nidx_v4, nrows_v, ns_v, sem):
    wid = lax.axis_index("s") * NC + lax.axis_index("c")
    base = wid * BW
    nidx_vs = (nidx_v0, nidx_v1, nidx_v2, nidx_v3, nidx_v4)

    for table, idx_src, out in ((emb_u_hbm, uidx_hbm, u_out),
                                (emb_v_hbm, vidx_hbm, v_out)):
        for c in range(BW // UCH):
            off = base + c * UCH
            pltpu.sync_copy(idx_src.at[pl.ds(off, UCH)], idx_v)
            pltpu.async_copy(table.at[idx_v], rows_v, sem).wait()
            pltpu.sync_copy(rows_v, out.at[pl.ds(off, UCH)])

    for c in range(BW // NCH):
        off = (base + c * NCH) * NEG
        for s in range(NSEG):
            pltpu.sync_copy(nidx_hbm.at[pl.ds(off + s * 128, 128)],
                            nidx_vs[s])
        cps = []
        for s in range(NSEG):
            cps.append(pltpu.async_copy(
                emb_v_hbm.at[nidx_vs[s]],
                nrows_v.at[pl.ds(s * 128, 128)], sem))
        for cp in cps:
            cp.wait()

        def acc_body(i, carry):
            r0 = i * NEG
            for d in range(D // 16):
                sl = pl.ds(d * 16, 16)
                s = nrows_v[r0, sl]
                for j in range(1, NEG):
                    s = s + nrows_v[r0 + j, sl]
                ns_v[i, sl] = s
            return carry

        lax.fori_loop(0, NCH, acc_body, 0)
        pltpu.sync_copy(ns_v, ns_out.at[pl.ds(base + c * NCH, NCH)])


@functools.lru_cache(maxsize=None)
def _make_sc_gather():
    return pl.kernel(
        _sc_gather_body,
        out_type=(
            jax.ShapeDtypeStruct((B, D), jnp.float32),
            jax.ShapeDtypeStruct((B, D), jnp.float32),
            jax.ShapeDtypeStruct((B, D), jnp.float32),
        ),
        mesh=plsc.VectorSubcoreMesh(core_axis_name="c", subcore_axis_name="s",
                                    num_cores=NC, num_subcores=NS),
        scratch_types=[
            pltpu.VMEM((UCH,), jnp.int32),
            pltpu.VMEM((UCH, D), jnp.float32),
            pltpu.VMEM((128,), jnp.int32),
            pltpu.VMEM((128,), jnp.int32),
            pltpu.VMEM((128,), jnp.int32),
            pltpu.VMEM((128,), jnp.int32),
            pltpu.VMEM((128,), jnp.int32),
            pltpu.VMEM((NROWS, D), jnp.float32),
            pltpu.VMEM((NCH, D), jnp.float32),
            pltpu.SemaphoreType.DMA,
        ],
    )


RB = 2048
NGRID = B // RB


def _tc_body(u_ref, v_ref, ns_ref, com_ref, cluster_ref, loss_ref, acc_ref):
    step = pl.program_id(0)

    @pl.when(step == 0)
    def _():
        acc_ref[0] = 0.0
        acc_ref[1] = 0.0

    u = u_ref[...]
    v = v_ref[...]
    ns = ns_ref[...]
    com = com_ref[...]

    pos = jnp.sum(u * v, axis=1)
    neg = -jnp.sum(u * ns, axis=1)
    ls = jax.nn.log_sigmoid(pos) + jax.nn.log_sigmoid(neg)

    dots = lax.dot_general(u, com, (((1,), (1,)), ((), ())),
                           preferred_element_type=jnp.float32,
                           precision=lax.Precision.HIGHEST)
    un2 = jnp.sum(u * u, axis=1)
    cn2 = jnp.sum(com * com, axis=1)
    d2 = un2[:, None] - 2.0 * dots + cn2[None, :]
    mind2 = jnp.min(d2, axis=1)
    ids = lax.broadcasted_iota(jnp.int32, (RB, K), 1)
    picked = jnp.where(d2 == mind2[:, None], ids, K)
    cluster_ref[...] = jnp.min(picked, axis=1)

    acc_ref[0] += jnp.sum(ls)
    acc_ref[1] += jnp.sum(jnp.maximum(mind2, 0.0))

    @pl.when(step == NGRID - 1)
    def _():
        loss_ref[0, 0] = -(acc_ref[0] / B) + GAMMA * (acc_ref[1] / B)


_tc_call = pl.pallas_call(
    _tc_body,
    grid=(NGRID,),
    in_specs=[
        pl.BlockSpec((RB, D), lambda i: (i, 0)),
        pl.BlockSpec((RB, D), lambda i: (i, 0)),
        pl.BlockSpec((RB, D), lambda i: (i, 0)),
        pl.BlockSpec((K, D), lambda i: (0, 0)),
    ],
    out_specs=[
        pl.BlockSpec((RB,), lambda i: (i,)),
        pl.BlockSpec(memory_space=pltpu.SMEM, block_shape=(1, 1),
                     index_map=lambda i: (0, 0)),
    ],
    out_shape=[
        jax.ShapeDtypeStruct((B,), jnp.int32),
        jax.ShapeDtypeStruct((1, 1), jnp.float32),
    ],
    scratch_shapes=[pltpu.SMEM((2,), jnp.float32)],
)


def kernel(u_node, v_node, negative_nodes, emb_u, emb_v, emb_com):
    u_idx = u_node.reshape(B).astype(jnp.int32)
    v_idx = v_node.reshape(B).astype(jnp.int32)
    n_idx = negative_nodes.reshape(B * NEG).astype(jnp.int32)

    u_emb, v_emb, neg_sum = _make_sc_gather()(u_idx, v_idx, n_idx, emb_u, emb_v)
    cluster, loss = _tc_call(u_emb, v_emb, neg_sum, emb_com)
    return (jnp.float32(GAMMA), loss.reshape(()), cluster)


# speedup vs baseline: 3.2874x; 1.5543x over previous
import functools
import jax
import jax.numpy as jnp
from jax import lax
from jax.experimental import pallas as pl
from jax.experimental.pallas import tpu as pltpu
from jax.experimental.pallas import tpu_sc as plsc

B = 16384
D = 128
NEG = 10
K = 64
GAMMA = 0.01

NC = 2
NS = 16
NW = NC * NS
BW = B // NW

CH = 32
NCHUNK = BW // CH
NROWS = CH * NEG
LANES = 16


def _fire(c, slot, uidx_v, vidx_v, nidx_v, emb_u_hbm, emb_v_hbm,
          urow, vrow, nrows, sems):
    cps = [
        pltpu.async_copy(emb_u_hbm.at[uidx_v.at[pl.ds(c * CH, CH)]],
                         urow[slot], sems[slot]),
        pltpu.async_copy(emb_v_hbm.at[vidx_v.at[pl.ds(c * CH, CH)]],
                         vrow[slot], sems[slot]),
    ]
    for s, ln in ((0, 128), (1, 128), (2, 64)):
        cps.append(pltpu.async_copy(
            emb_v_hbm.at[nidx_v.at[pl.ds(c * NROWS + s * 128, ln)]],
            nrows[slot].at[pl.ds(s * 128, ln)], sems[slot]))
    return cps


def _sc_body(uidx_hbm, vidx_hbm, nidx_hbm, emb_u_hbm, emb_v_hbm,
             u_out, parts_out,
             uidx_v, vidx_v, nidx_v, urow0, urow1, vrow0, vrow1,
             nrows0, nrows1, parts_v, sem0, sem1):
    wid = lax.axis_index("s") * NC + lax.axis_index("c")
    base = wid * BW
    urow = (urow0, urow1)
    vrow = (vrow0, vrow1)
    nrows = (nrows0, nrows1)
    sems = (sem0, sem1)

    pltpu.sync_copy(uidx_hbm.at[pl.ds(base, BW)], uidx_v)
    pltpu.sync_copy(vidx_hbm.at[pl.ds(base, BW)], vidx_v)
    pltpu.sync_copy(nidx_hbm.at[pl.ds(base * NEG, BW * NEG)], nidx_v)

    inflight = _fire(0, 0, uidx_v, vidx_v, nidx_v, emb_u_hbm, emb_v_hbm,
                     urow, vrow, nrows, sems)

    for c in range(NCHUNK):
        slot = c % 2
        nxt = inflight
        if c + 1 < NCHUNK:
            inflight = _fire(c + 1, 1 - slot, uidx_v, vidx_v, nidx_v,
                             emb_u_hbm, emb_v_hbm, urow, vrow, nrows, sems)
        for cp in nxt:
            cp.wait()

        ur = urow[slot]
        vr = vrow[slot]
        nr = nrows[slot]

        def item_body(i, carry, ur=ur, vr=vr, nr=nr):
            r0 = i * NEG
            pa = None
            na = None
            for d in range(D // 16):
                sl = pl.ds(d * 16, 16)
                ud = ur[i, sl]
                nsd = nr[r0, sl]
                for j in range(1, NEG):
                    nsd = nsd + nr[r0 + j, sl]
                pd = ud * vr[i, sl]
                nd = ud * nsd
                pa = pd if pa is None else pa + pd
                na = nd if na is None else na + nd
            parts_v[i, pl.ds(0, 16)] = pa
            parts_v[i, pl.ds(16, 16)] = -na
            return carry

        lax.fori_loop(0, CH, item_body, 0)

        pltpu.sync_copy(ur, u_out.at[pl.ds(base + c * CH, CH)])
        pltpu.sync_copy(parts_v, parts_out.at[pl.ds(base + c * CH, CH)])


@functools.lru_cache(maxsize=None)
def _make_sc_main():
    return pl.kernel(
        _sc_body,
        out_type=(
            jax.ShapeDtypeStruct((B, D), jnp.float32),
            jax.ShapeDtypeStruct((B, 32), jnp.float32),
        ),
        mesh=plsc.VectorSubcoreMesh(core_axis_name="c", subcore_axis_name="s",
                                    num_cores=NC, num_subcores=NS),
        scratch_types=[
            pltpu.VMEM((BW,), jnp.int32),
            pltpu.VMEM((BW,), jnp.int32),
            pltpu.VMEM((BW * NEG,), jnp.int32),
            pltpu.VMEM((CH, D), jnp.float32),
            pltpu.VMEM((CH, D), jnp.float32),
            pltpu.VMEM((CH, D), jnp.float32),
            pltpu.VMEM((CH, D), jnp.float32),
            pltpu.VMEM((NROWS, D), jnp.float32),
            pltpu.VMEM((NROWS, D), jnp.float32),
            pltpu.VMEM((CH, 32), jnp.float32),
            pltpu.SemaphoreType.DMA,
            pltpu.SemaphoreType.DMA,
        ],
    )


RB = 2048
NGRID = B // RB


def _tc_body(u_ref, parts_ref, com_ref, cluster_ref, loss_ref, acc_ref):
    step = pl.program_id(0)

    @pl.when(step == 0)
    def _():
        acc_ref[0] = 0.0
        acc_ref[1] = 0.0

    u = u_ref[...]
    com = com_ref[...]

    gsel = (lax.broadcasted_iota(jnp.int32, (32, 2), 0) // 16
            == lax.broadcasted_iota(jnp.int32, (32, 2), 1))
    scores = lax.dot_general(parts_ref[...], gsel.astype(jnp.float32),
                             (((1,), (0,)), ((), ())),
                             preferred_element_type=jnp.float32,
                             precision=lax.Precision.HIGHEST)
    ls = jax.nn.log_sigmoid(scores)

    dots = lax.dot_general(u, com, (((1,), (1,)), ((), ())),
                           preferred_element_type=jnp.float32,
                           precision=lax.Precision.HIGHEST)
    un2 = jnp.sum(u * u, axis=1)
    cn2 = jnp.sum(com * com, axis=1)
    d2 = un2[:, None] - 2.0 * dots + cn2[None, :]
    mind2 = jnp.min(d2, axis=1)
    ids = lax.broadcasted_iota(jnp.int32, (RB, K), 1)
    picked = jnp.where(d2 == mind2[:, None], ids, K)
    cluster_ref[...] = jnp.min(picked, axis=1)

    acc_ref[0] += jnp.sum(ls)
    acc_ref[1] += jnp.sum(jnp.maximum(mind2, 0.0))

    @pl.when(step == NGRID - 1)
    def _():
        loss_ref[0, 0] = -(acc_ref[0] / B) + GAMMA * (acc_ref[1] / B)


_tc_call = pl.pallas_call(
    _tc_body,
    grid=(NGRID,),
    in_specs=[
        pl.BlockSpec((RB, D), lambda i: (i, 0)),
        pl.BlockSpec((RB, 32), lambda i: (i, 0)),
        pl.BlockSpec((K, D), lambda i: (0, 0)),
    ],
    out_specs=[
        pl.BlockSpec((RB,), lambda i: (i,)),
        pl.BlockSpec(memory_space=pltpu.SMEM, block_shape=(1, 1),
                     index_map=lambda i: (0, 0)),
    ],
    out_shape=[
        jax.ShapeDtypeStruct((B,), jnp.int32),
        jax.ShapeDtypeStruct((1, 1), jnp.float32),
    ],
    scratch_shapes=[pltpu.SMEM((2,), jnp.float32)],
)


def kernel(u_node, v_node, negative_nodes, emb_u, emb_v, emb_com):
    u_idx = u_node.reshape(B).astype(jnp.int32)
    v_idx = v_node.reshape(B).astype(jnp.int32)
    n_idx = negative_nodes.reshape(B * NEG).astype(jnp.int32)

    u_emb, parts = _make_sc_main()(u_idx, v_idx, n_idx, emb_u, emb_v)
    cluster, loss = _tc_call(u_emb, parts, emb_com)
    return (jnp.float32(GAMMA), loss.reshape(()), cluster)


# speedup vs baseline: 3.9545x; 1.2029x over previous
import functools
import jax
import jax.numpy as jnp
from jax import lax
from jax.experimental import pallas as pl
from jax.experimental.pallas import tpu as pltpu
from jax.experimental.pallas import tpu_sc as plsc

B = 16384
D = 128
NEG = 10
K = 64
GAMMA = 0.01

NC = 2
NS = 16
NW = NC * NS
BW = B // NW

CH = 32
NCHUNK = BW // CH
NROWS = CH * NEG
LANES = 16


def _fire(c, slot, uidx_v, vidx_v, nidx_v, emb_u_hbm, emb_v_hbm,
          urow, vrow, nrows, sems):
    cps = [
        pltpu.async_copy(emb_u_hbm.at[uidx_v.at[pl.ds(c * CH, CH)]],
                         urow[slot], sems[slot]),
        pltpu.async_copy(emb_v_hbm.at[vidx_v.at[pl.ds(c * CH, CH)]],
                         vrow[slot], sems[slot]),
    ]
    for s, ln in ((0, 128), (1, 128), (2, 64)):
        cps.append(pltpu.async_copy(
            emb_v_hbm.at[nidx_v.at[pl.ds(c * NROWS + s * 128, ln)]],
            nrows[slot].at[pl.ds(s * 128, ln)], sems[slot]))
    return cps


def _sc_body(uidx_hbm, vidx_hbm, nidx_hbm, emb_u_hbm, emb_v_hbm,
             u_out, parts_out,
             uidx_v, vidx_v, nidx_v, urow0, urow1, vrow0, vrow1,
             nrows0, nrows1, parts_v, sem0, sem1):
    wid = lax.axis_index("s") * NC + lax.axis_index("c")
    base = wid * BW
    urow = (urow0, urow1)
    vrow = (vrow0, vrow1)
    nrows = (nrows0, nrows1)
    sems = (sem0, sem1)

    pltpu.sync_copy(uidx_hbm.at[pl.ds(base, BW)], uidx_v)
    pltpu.sync_copy(vidx_hbm.at[pl.ds(base, BW)], vidx_v)
    pltpu.sync_copy(nidx_hbm.at[pl.ds(base * NEG, BW * NEG)], nidx_v)

    inflight = _fire(0, 0, uidx_v, vidx_v, nidx_v, emb_u_hbm, emb_v_hbm,
                     urow, vrow, nrows, sems)

    for c in range(NCHUNK):
        slot = c % 2
        nxt = inflight
        if c + 1 < NCHUNK:
            inflight = _fire(c + 1, 1 - slot, uidx_v, vidx_v, nidx_v,
                             emb_u_hbm, emb_v_hbm, urow, vrow, nrows, sems)
        for cp in nxt:
            cp.wait()

        ur = urow[slot]
        vr = vrow[slot]
        nr = nrows[slot]

        def item_body(i, carry, ur=ur, vr=vr, nr=nr):
            r0 = i * NEG
            pa = None
            na = None
            for d in range(D // 16):
                sl = pl.ds(d * 16, 16)
                ud = ur[i, sl]
                nsd = nr[r0, sl]
                for j in range(1, NEG):
                    nsd = nsd + nr[r0 + j, sl]
                pd = ud * vr[i, sl]
                nd = ud * nsd
                pa = pd if pa is None else pa + pd
                na = nd if na is None else na + nd
            parts_v[i, pl.ds(0, 16)] = pa
            parts_v[i, pl.ds(16, 16)] = -na
            return carry

        lax.fori_loop(0, CH, item_body, 0)

        pltpu.sync_copy(ur, u_out.at[pl.ds(base + c * CH, CH)])
        pltpu.sync_copy(parts_v, parts_out.at[pl.ds(base + c * CH, CH)])


@functools.lru_cache(maxsize=None)
def _make_sc_main():
    return pl.kernel(
        _sc_body,
        out_type=(
            jax.ShapeDtypeStruct((B, D), jnp.float32),
            jax.ShapeDtypeStruct((B, 32), jnp.float32),
        ),
        mesh=plsc.VectorSubcoreMesh(core_axis_name="c", subcore_axis_name="s",
                                    num_cores=NC, num_subcores=NS),
        scratch_types=[
            pltpu.VMEM((BW,), jnp.int32),
            pltpu.VMEM((BW,), jnp.int32),
            pltpu.VMEM((BW * NEG,), jnp.int32),
            pltpu.VMEM((CH, D), jnp.float32),
            pltpu.VMEM((CH, D), jnp.float32),
            pltpu.VMEM((CH, D), jnp.float32),
            pltpu.VMEM((CH, D), jnp.float32),
            pltpu.VMEM((NROWS, D), jnp.float32),
            pltpu.VMEM((NROWS, D), jnp.float32),
            pltpu.VMEM((CH, 32), jnp.float32),
            pltpu.SemaphoreType.DMA,
            pltpu.SemaphoreType.DMA,
        ],
    )


RB = 4096
NGRID = B // RB


def _tc_body(u_ref, parts_ref, com_ref, cluster_ref, loss_ref, acc_ref):
    step = pl.program_id(0)

    @pl.when(step == 0)
    def _():
        acc_ref[0] = 0.0
        acc_ref[1] = 0.0

    u = u_ref[...]
    com = com_ref[...]

    gsel = (lax.broadcasted_iota(jnp.int32, (32, 2), 0) // 16
            == lax.broadcasted_iota(jnp.int32, (32, 2), 1))
    scores = lax.dot_general(parts_ref[...], gsel.astype(jnp.float32),
                             (((1,), (0,)), ((), ())),
                             preferred_element_type=jnp.float32,
                             precision=lax.Precision.HIGHEST)
    ls = jax.nn.log_sigmoid(scores)

    dots = lax.dot_general(com, u, (((1,), (1,)), ((), ())),
                           preferred_element_type=jnp.float32,
                           precision=lax.Precision.HIGHEST)
    cn2 = jnp.sum(com * com, axis=1)
    m = cn2[:, None] - 2.0 * dots
    ii = lax.broadcasted_iota(jnp.int32, (K, RB), 0)
    for half in (32, 16, 8, 4, 2, 1):
        tm, bm = m[:half], m[half:]
        ti, bi = ii[:half], ii[half:]
        tk = bm < tm
        m = jnp.where(tk, bm, tm)
        ii = jnp.where(tk, bi, ti)
    cluster_ref[...] = ii[0]

    acc_ref[0] += jnp.sum(ls)
    acc_ref[1] += jnp.sum(m) + jnp.sum(u * u)

    @pl.when(step == NGRID - 1)
    def _():
        loss_ref[0, 0] = -(acc_ref[0] / B) + GAMMA * (acc_ref[1] / B)


_tc_call = pl.pallas_call(
    _tc_body,
    grid=(NGRID,),
    in_specs=[
        pl.BlockSpec((RB, D), lambda i: (i, 0)),
        pl.BlockSpec((RB, 32), lambda i: (i, 0)),
        pl.BlockSpec((K, D), lambda i: (0, 0)),
    ],
    out_specs=[
        pl.BlockSpec((RB,), lambda i: (i,)),
        pl.BlockSpec(memory_space=pltpu.SMEM, block_shape=(1, 1),
                     index_map=lambda i: (0, 0)),
    ],
    out_shape=[
        jax.ShapeDtypeStruct((B,), jnp.int32),
        jax.ShapeDtypeStruct((1, 1), jnp.float32),
    ],
    scratch_shapes=[pltpu.SMEM((2,), jnp.float32)],
)


def kernel(u_node, v_node, negative_nodes, emb_u, emb_v, emb_com):
    u_idx = u_node.reshape(B).astype(jnp.int32)
    v_idx = v_node.reshape(B).astype(jnp.int32)
    n_idx = negative_nodes.reshape(B * NEG).astype(jnp.int32)

    u_emb, parts = _make_sc_main()(u_idx, v_idx, n_idx, emb_u, emb_v)
    cluster, loss = _tc_call(u_emb, parts, emb_com)
    return (jnp.float32(GAMMA), loss.reshape(()), cluster)


# speedup vs baseline: 4.3599x; 1.1025x over previous
import functools
import jax
import jax.numpy as jnp
from jax import lax
from jax.experimental import pallas as pl
from jax.experimental.pallas import tpu as pltpu
from jax.experimental.pallas import tpu_sc as plsc

B = 16384
D = 128
NEG = 10
K = 64
GAMMA = 0.01

NC = 2
NS = 16
NW = NC * NS
BW = B // NW

CH = 32
NCHUNK = BW // CH
NROWS = CH * NEG
LANES = 16


def _fire(c, slot, uidx_v, vidx_v, nidx_v, emb_u_hbm, emb_v_hbm,
          urow, vrow, nrows, sems):
    cps = [
        pltpu.async_copy(emb_u_hbm.at[uidx_v.at[pl.ds(c * CH, CH)]],
                         urow[slot], sems[slot]),
        pltpu.async_copy(emb_v_hbm.at[vidx_v.at[pl.ds(c * CH, CH)]],
                         vrow[slot], sems[slot]),
    ]
    for s, ln in ((0, 128), (1, 128), (2, 64)):
        cps.append(pltpu.async_copy(
            emb_v_hbm.at[nidx_v.at[pl.ds(c * NROWS + s * 128, ln)]],
            nrows[slot].at[pl.ds(s * 128, ln)], sems[slot]))
    return cps


def _sc_body(uidx_hbm, vidx_hbm, nidx_hbm, emb_u_hbm, emb_v_hbm,
             u_out, parts_out,
             uidx_v, vidx_v, nidx_v, urow0, urow1, vrow0, vrow1,
             nrows0, nrows1, parts_v0, parts_v1, sem0, sem1, wsem0, wsem1):
    wid = lax.axis_index("s") * NC + lax.axis_index("c")
    base = wid * BW
    urow = (urow0, urow1)
    vrow = (vrow0, vrow1)
    nrows = (nrows0, nrows1)
    parts = (parts_v0, parts_v1)
    sems = (sem0, sem1)
    wsems = (wsem0, wsem1)

    pltpu.sync_copy(uidx_hbm.at[pl.ds(base, BW)], uidx_v)
    pltpu.sync_copy(vidx_hbm.at[pl.ds(base, BW)], vidx_v)
    pltpu.sync_copy(nidx_hbm.at[pl.ds(base * NEG, BW * NEG)], nidx_v)

    inflight = _fire(0, 0, uidx_v, vidx_v, nidx_v, emb_u_hbm, emb_v_hbm,
                     urow, vrow, nrows, sems)
    pending = [(), ()]

    for c in range(NCHUNK):
        slot = c % 2
        nxt = inflight
        if c + 1 < NCHUNK:
            for wp in pending[1 - slot]:
                wp.wait()
            pending[1 - slot] = ()
            inflight = _fire(c + 1, 1 - slot, uidx_v, vidx_v, nidx_v,
                             emb_u_hbm, emb_v_hbm, urow, vrow, nrows, sems)
        for cp in nxt:
            cp.wait()

        ur = urow[slot]
        vr = vrow[slot]
        nr = nrows[slot]
        parts_v = parts[slot]

        def item_body(i, carry, ur=ur, vr=vr, nr=nr):
            r0 = i * NEG
            pa = None
            na = None
            for d in range(D // 16):
                sl = pl.ds(d * 16, 16)
                ud = ur[i, sl]
                nsd = nr[r0, sl]
                for j in range(1, NEG):
                    nsd = nsd + nr[r0 + j, sl]
                pd = ud * vr[i, sl]
                nd = ud * nsd
                pa = pd if pa is None else pa + pd
                na = nd if na is None else na + nd
            parts_v[i, pl.ds(0, 16)] = pa
            parts_v[i, pl.ds(16, 16)] = -na
            return carry

        lax.fori_loop(0, CH, item_body, 0)

        for wp in pending[slot]:
            wp.wait()
        pending[slot] = (
            pltpu.async_copy(ur, u_out.at[pl.ds(base + c * CH, CH)],
                             wsems[slot]),
            pltpu.async_copy(parts_v,
                             parts_out.at[pl.ds(base + c * CH, CH)],
                             wsems[slot]),
        )

    for ps in pending:
        for wp in ps:
            wp.wait()


@functools.lru_cache(maxsize=None)
def _make_sc_main():
    return pl.kernel(
        _sc_body,
        out_type=(
            jax.ShapeDtypeStruct((B, D), jnp.float32),
            jax.ShapeDtypeStruct((B, 32), jnp.float32),
        ),
        mesh=plsc.VectorSubcoreMesh(core_axis_name="c", subcore_axis_name="s",
                                    num_cores=NC, num_subcores=NS),
        scratch_types=[
            pltpu.VMEM((BW,), jnp.int32),
            pltpu.VMEM((BW,), jnp.int32),
            pltpu.VMEM((BW * NEG,), jnp.int32),
            pltpu.VMEM((CH, D), jnp.float32),
            pltpu.VMEM((CH, D), jnp.float32),
            pltpu.VMEM((CH, D), jnp.float32),
            pltpu.VMEM((CH, D), jnp.float32),
            pltpu.VMEM((NROWS, D), jnp.float32),
            pltpu.VMEM((NROWS, D), jnp.float32),
            pltpu.VMEM((CH, 32), jnp.float32),
            pltpu.VMEM((CH, 32), jnp.float32),
            pltpu.SemaphoreType.DMA,
            pltpu.SemaphoreType.DMA,
            pltpu.SemaphoreType.DMA,
            pltpu.SemaphoreType.DMA,
        ],
    )


RB = 4096
NGRID = B // RB


def _tc_body(u_ref, parts_ref, com_ref, cluster_ref, loss_ref, acc_ref):
    step = pl.program_id(0)

    @pl.when(step == 0)
    def _():
        acc_ref[0] = 0.0
        acc_ref[1] = 0.0

    u = u_ref[...]
    com = com_ref[...]

    gsel = (lax.broadcasted_iota(jnp.int32, (2, 32), 0)
            == lax.broadcasted_iota(jnp.int32, (2, 32), 1) // 16)
    scores = lax.dot_general(gsel.astype(jnp.float32), parts_ref[...],
                             (((1,), (1,)), ((), ())),
                             preferred_element_type=jnp.float32)
    ls = jax.nn.log_sigmoid(scores)

    dots = lax.dot_general(com, u, (((1,), (1,)), ((), ())),
                           preferred_element_type=jnp.float32,
                           precision=lax.Precision.HIGHEST)
    cn2 = jnp.sum(com * com, axis=1)
    m = cn2[:, None] - 2.0 * dots
    ii = lax.broadcasted_iota(jnp.int32, (K, RB), 0)
    for half in (32, 16, 8, 4, 2, 1):
        tm, bm = m[:half], m[half:]
        ti, bi = ii[:half], ii[half:]
        tk = bm < tm
        m = jnp.where(tk, bm, tm)
        ii = jnp.where(tk, bi, ti)
    cluster_ref[...] = ii[0]

    acc_ref[0] += jnp.sum(ls)
    acc_ref[1] += jnp.sum(m) + jnp.sum(u * u)

    @pl.when(step == NGRID - 1)
    def _():
        loss_ref[0, 0] = -(acc_ref[0] / B) + GAMMA * (acc_ref[1] / B)


_tc_call = pl.pallas_call(
    _tc_body,
    grid=(NGRID,),
    in_specs=[
        pl.BlockSpec((RB, D), lambda i: (i, 0)),
        pl.BlockSpec((RB, 32), lambda i: (i, 0)),
        pl.BlockSpec((K, D), lambda i: (0, 0)),
    ],
    out_specs=[
        pl.BlockSpec((RB,), lambda i: (i,)),
        pl.BlockSpec(memory_space=pltpu.SMEM, block_shape=(1, 1),
                     index_map=lambda i: (0, 0)),
    ],
    out_shape=[
        jax.ShapeDtypeStruct((B,), jnp.int32),
        jax.ShapeDtypeStruct((1, 1), jnp.float32),
    ],
    scratch_shapes=[pltpu.SMEM((2,), jnp.float32)],
)


def kernel(u_node, v_node, negative_nodes, emb_u, emb_v, emb_com):
    u_idx = u_node.reshape(B).astype(jnp.int32)
    v_idx = v_node.reshape(B).astype(jnp.int32)
    n_idx = negative_nodes.reshape(B * NEG).astype(jnp.int32)

    u_emb, parts = _make_sc_main()(u_idx, v_idx, n_idx, emb_u, emb_v)
    cluster, loss = _tc_call(u_emb, parts, emb_com)
    return (jnp.float32(GAMMA), loss.reshape(()), cluster)
